# baseline (speedup 1.0000x reference)
import functools

import jax
import jax.numpy as jnp
from jax import lax
from jax.experimental import pallas as pl
from jax.experimental.pallas import tpu as pltpu
from jax.experimental.pallas import tpu_sc as plsc

_B = 16384
_D = 3
_V = 1000000
_L = 16

_info = plsc.get_sparse_core_info()
_NC, _NS = _info.num_cores, _info.num_subcores
_NW = _NC * _NS
_BPW = _B // _NW
_EPW = _BPW * _D
_NG = _BPW // _L
_CHUNK = _D * 2 * _L
_NCHUNK = _EPW // _CHUNK

_mesh = plsc.VectorSubcoreMesh(core_axis_name="c", subcore_axis_name="s")


@functools.partial(
    pl.kernel,
    mesh=_mesh,
    compiler_params=pltpu.CompilerParams(
        use_tc_tiling_on_sc=False, needs_layout_passes=False
    ),
    out_type=jax.ShapeDtypeStruct((_B * _D,), jnp.float32),
    scratch_types=[
        pltpu.VMEM((_BPW,), jnp.int32),
        [pltpu.VMEM((_CHUNK,), jnp.int32) for _ in range(_NCHUNK)],
        [pltpu.VMEM((_CHUNK,), jnp.float32) for _ in range(_NCHUNK)],
        pltpu.SemaphoreType.DMA,
    ],
)
def _gather_kernel(idx_hbm, table_hbm, out_hbm, idx_v, eidx_vs, vals_vs, sem):
    wid = lax.axis_index("s") * _NC + lax.axis_index("c")
    pltpu.sync_copy(idx_hbm.at[pl.ds(wid * _BPW, _BPW)], idx_v)

    lanes = jnp.arange(_L, dtype=jnp.int32)
    for g in range(_NG):
        v = idx_v[pl.ds(_L * g, _L)]
        t = v * 3
        base = (_D * _L) * (g % 2) + _D * lanes
        for c in range(_D):
            plsc.store_scatter(eidx_vs[g // 2], [base + c], t + c)

    copies = [
        pltpu.async_copy(table_hbm.at[eidx_vs[j]], vals_vs[j], sem)
        for j in range(_NCHUNK)
    ]
    for c in copies:
        c.wait()

    for j in range(_NCHUNK):
        pltpu.sync_copy(
            vals_vs[j], out_hbm.at[pl.ds(wid * _EPW + j * _CHUNK, _CHUNK)]
        )


def kernel(points, delta_translation):
    flat = _gather_kernel(points, delta_translation.reshape(_V * _D))
    return flat.reshape(_B, _D)


# speedup vs baseline: 12.6132x; 12.6132x over previous
import functools

import jax
import jax.numpy as jnp
from jax import lax
from jax.experimental import pallas as pl
from jax.experimental.pallas import tpu as pltpu
from jax.experimental.pallas import tpu_sc as plsc

_B = 16384
_D = 3
_V = 1000000
_L = 16

_info = plsc.get_sparse_core_info()
_NC, _NS = _info.num_cores, _info.num_subcores
_NW = _NC * _NS
_BPW = _B // _NW

_mesh = plsc.VectorSubcoreMesh(core_axis_name="c", subcore_axis_name="s")


@functools.partial(
    pl.kernel,
    mesh=_mesh,
    out_type=jax.ShapeDtypeStruct((_B, _D), jnp.float32),
    scratch_types=[
        pltpu.VMEM((_BPW,), jnp.int32),
        pltpu.VMEM((_BPW, _D), jnp.float32),
        pltpu.SemaphoreType.DMA,
    ],
)
def _gather_kernel(idx_hbm, table_hbm, out_hbm, idx_v, rows_v, sem):
    wid = lax.axis_index("s") * _NC + lax.axis_index("c")
    pltpu.sync_copy(idx_hbm.at[pl.ds(wid * _BPW, _BPW)], idx_v)

    for g in range(_BPW // _L):
        v = idx_v[pl.ds(_L * g, _L)]
        for lane in range(_L):
            pltpu.async_copy(
                table_hbm.at[v[lane]], rows_v.at[_L * g + lane], sem
            )
    pltpu.make_async_copy(
        table_hbm.at[pl.ds(0, _BPW)], rows_v, sem
    ).wait()

    pltpu.sync_copy(rows_v, out_hbm.at[pl.ds(wid * _BPW, _BPW)])


def kernel(points, delta_translation):
    return _gather_kernel(points, delta_translation)
